# Initial kernel scaffold; baseline (speedup 1.0000x reference)
import functools

import jax
import jax.numpy as jnp
from jax import lax
from jax.experimental import pallas as pl
from jax.experimental.pallas import tpu as pltpu
from jax.experimental.pallas import tpu_sc as plsc

NC = 2
NS = 16
NW = NC * NS
SUB = 80


def _node_proj_body(x_ref, w_ref, b_ref, xs1_ref, xd1_ref, xm_ref, hb_ref):
    p = jnp.dot(x_ref[...], w_ref[...], preferred_element_type=jnp.float32)
    xs1_ref[...] = p[:, 0:16]
    xd1_ref[...] = p[:, 16:32]
    xm_ref[...] = p[:, 32:160]
    hb_ref[...] = p[:, 160:288] + b_ref[...]


def _node_proj(x, wcat, bself2):
    n, dv = x.shape
    blk = 2000
    return pl.pallas_call(
        _node_proj_body,
        grid=(n // blk,),
        in_specs=[
            pl.BlockSpec((blk, dv), lambda i: (i, 0)),
            pl.BlockSpec(wcat.shape, lambda i: (0, 0)),
            pl.BlockSpec((1, dv), lambda i: (0, 0)),
        ],
        out_specs=[
            pl.BlockSpec((blk, 16), lambda i: (i, 0)),
            pl.BlockSpec((blk, 16), lambda i: (i, 0)),
            pl.BlockSpec((blk, 128), lambda i: (i, 0)),
            pl.BlockSpec((blk, 128), lambda i: (i, 0)),
        ],
        out_shape=[
            jax.ShapeDtypeStruct((n, 16), jnp.float32),
            jax.ShapeDtypeStruct((n, 16), jnp.float32),
            jax.ShapeDtypeStruct((n, 128), jnp.float32),
            jax.ShapeDtypeStruct((n, 128), jnp.float32),
        ],
    )(x, wcat, bself2)


def _sc_gather_make(n, e):
    rows = e // SUB
    wrows = rows // NW
    jrows = 25
    nchunk = wrows // jrows
    mesh = plsc.VectorSubcoreMesh(core_axis_name="c", subcore_axis_name="s")

    @functools.partial(
        pl.kernel,
        out_type=jax.ShapeDtypeStruct((e, 16), jnp.float32),
        mesh=mesh,
        scratch_types=[
            pltpu.VMEM((jrows, SUB), jnp.int32),
            pltpu.VMEM((jrows, SUB), jnp.int32),
            pltpu.VMEM((jrows * SUB, 16), jnp.float32),
            pltpu.VMEM((jrows * SUB, 16), jnp.float32),
            pltpu.SemaphoreType.DMA,
        ],
    )
    def k(xs1_hbm, xd1_hbm, src2_hbm, dst2_hbm, out_hbm,
          idxs_v, idxd_v, g1_v, g2_v, sem):
        wid = lax.axis_index("c") * NS + lax.axis_index("s")
        row0 = wid * wrows

        def chunk(ci, carry):
            r0 = row0 + ci * jrows
            pltpu.sync_copy(src2_hbm.at[pl.ds(r0, jrows)], idxs_v)
            pltpu.sync_copy(dst2_hbm.at[pl.ds(r0, jrows)], idxd_v)
            cps = []
            for j in range(jrows):
                cps.append(pltpu.async_copy(
                    xs1_hbm.at[idxs_v.at[j]],
                    g1_v.at[pl.ds(j * SUB, SUB)], sem))
                cps.append(pltpu.async_copy(
                    xd1_hbm.at[idxd_v.at[j]],
                    g2_v.at[pl.ds(j * SUB, SUB)], sem))
            for cp in cps:
                cp.wait()

            def addrow(i, c2):
                g1_v[i, :] = g1_v[i, :] + g2_v[i, :]
                return c2
            lax.fori_loop(0, jrows * SUB, addrow, 0)
            pltpu.sync_copy(g1_v, out_hbm.at[pl.ds(r0 * SUB, jrows * SUB)])
            return carry

        lax.fori_loop(0, nchunk, chunk, 0)

    return k


def _edge_mlp_body(gs_ref, ea_ref, w1_ref, b1_ref, w2_ref, b2_ref,
                   wm_ref, bm_ref, e_ref, t_ref):
    pre = (gs_ref[...]
           + jnp.dot(ea_ref[...], w1_ref[...], preferred_element_type=jnp.float32)
           + b1_ref[...])
    r = jnp.maximum(pre, 0.0)
    e = jnp.dot(r, w2_ref[...], preferred_element_type=jnp.float32) + b2_ref[...]
    e_ref[...] = e
    t_ref[...] = (jnp.dot(e, wm_ref[...], preferred_element_type=jnp.float32)
                  + bm_ref[...])


def _edge_mlp(gsum, ea, w1, b1, w2, b2, wm, bm):
    e_n, de = ea.shape
    blk = 4000
    return pl.pallas_call(
        _edge_mlp_body,
        grid=(e_n // blk,),
        in_specs=[
            pl.BlockSpec((blk, 16), lambda i: (i, 0)),
            pl.BlockSpec((blk, de), lambda i: (i, 0)),
            pl.BlockSpec(w1.shape, lambda i: (0, 0)),
            pl.BlockSpec((1, 16), lambda i: (0, 0)),
            pl.BlockSpec(w2.shape, lambda i: (0, 0)),
            pl.BlockSpec((1, 16), lambda i: (0, 0)),
            pl.BlockSpec(wm.shape, lambda i: (0, 0)),
            pl.BlockSpec((1, 128), lambda i: (0, 0)),
        ],
        out_specs=[
            pl.BlockSpec((blk, 16), lambda i: (i, 0)),
            pl.BlockSpec((blk, 128), lambda i: (i, 0)),
        ],
        out_shape=[
            jax.ShapeDtypeStruct((e_n, 16), jnp.float32),
            jax.ShapeDtypeStruct((e_n, 128), jnp.float32),
        ],
    )(gsum, ea, w1, b1, w2, b2, wm, bm)


def _sc_msg_make(n, e):
    rows = e // SUB
    wrows = rows // NW
    jrows = 5
    nchunk = wrows // jrows
    npc = n // NS
    mesh = plsc.VectorSubcoreMesh(core_axis_name="c", subcore_axis_name="s")

    @functools.partial(
        pl.kernel,
        out_type=jax.ShapeDtypeStruct((NC, n, 128), jnp.float32),
        mesh=mesh,
        scratch_types=[
            pltpu.VMEM((jrows, SUB), jnp.int32),
            pltpu.VMEM((jrows, SUB), jnp.int32),
            pltpu.VMEM((jrows * SUB, 128), jnp.float32),
            pltpu.VMEM((jrows * SUB, 128), jnp.float32),
            pltpu.VMEM_SHARED((n, 128), jnp.float32),
            pltpu.SemaphoreType.DMA,
        ],
    )
    def k(xm_hbm, t_hbm, src2_hbm, dst2_hbm, zeros_hbm, out_hbm,
          idxs_v, idxd_v, g_v, t_v, agg_sp, sem):
        c = lax.axis_index("c")
        s = lax.axis_index("s")
        wid = c * NS + s
        pltpu.sync_copy(zeros_hbm.at[pl.ds(s * npc, npc)],
                        agg_sp.at[pl.ds(s * npc, npc)])
        plsc.subcore_barrier()
        row0 = wid * wrows

        def chunk(ci, carry):
            r0 = row0 + ci * jrows
            pltpu.sync_copy(src2_hbm.at[pl.ds(r0, jrows)], idxs_v)
            pltpu.sync_copy(dst2_hbm.at[pl.ds(r0, jrows)], idxd_v)
            cps = [pltpu.async_copy(xm_hbm.at[idxs_v.at[j]],
                                    g_v.at[pl.ds(j * SUB, SUB)], sem)
                   for j in range(jrows)]
            pltpu.sync_copy(t_hbm.at[pl.ds(r0 * SUB, jrows * SUB)], t_v)
            for cp in cps:
                cp.wait()

            def addrow(i, c2):
                for jj in range(8):
                    sl = pl.ds(jj * 16, 16)
                    g_v[i, sl] = jnp.maximum(g_v[i, sl] + t_v[i, sl], 0.0)
                return c2
            lax.fori_loop(0, jrows * SUB, addrow, 0)
            for j in range(jrows):
                pltpu.sync_copy(g_v.at[pl.ds(j * SUB, SUB)],
                                agg_sp.at[idxd_v.at[j]], add=True)
            return carry

        lax.fori_loop(0, nchunk, chunk, 0)
        plsc.subcore_barrier()
        pltpu.sync_copy(agg_sp.at[pl.ds(s * npc, npc)],
                        out_hbm.at[c, pl.ds(s * npc, npc)])

    return k


def _finish_body(hb_ref, a0_ref, a1_ref, h_ref):
    h_ref[...] = jnp.maximum(hb_ref[...] + a0_ref[...] + a1_ref[...], 0.0)


def _finish(hbase, a0, a1):
    n, dv = hbase.shape
    blk = 2000
    spec = pl.BlockSpec((blk, dv), lambda i: (i, 0))
    return pl.pallas_call(
        _finish_body,
        grid=(n // blk,),
        in_specs=[spec, spec, spec],
        out_specs=spec,
        out_shape=jax.ShapeDtypeStruct((n, dv), jnp.float32),
    )(hbase, a0, a1)


def kernel(x, edge_index, edge_attr, We1, be1, We2, be2,
           Wmsg, bmsg, Wself, bself):
    n, dv = x.shape
    e = edge_index.shape[1]
    src2 = edge_index[0].reshape(e // SUB, SUB)
    dst2 = edge_index[1].reshape(e // SUB, SUB)

    wcat = jnp.concatenate(
        [We1[:dv], We1[dv:2 * dv], Wmsg[:dv], Wself], axis=1)
    xs1, xd1, xm, hbase = _node_proj(x, wcat, bself[None, :])

    gsum = _sc_gather_make(n, e)(xs1, xd1, src2, dst2)
    e_out, t = _edge_mlp(gsum, edge_attr, We1[2 * dv:], be1[None, :],
                         We2, be2[None, :], Wmsg[dv:], bmsg[None, :])
    agg2 = _sc_msg_make(n, e)(xm, t, src2, dst2,
                              jnp.zeros((n, 128), jnp.float32))
    h = _finish(hbase, agg2[0], agg2[1])
    return (h, e_out)


# speedup vs baseline: 2.6081x; 2.6081x over previous
import functools

import jax
import jax.numpy as jnp
from jax import lax
from jax.experimental import pallas as pl
from jax.experimental.pallas import tpu as pltpu
from jax.experimental.pallas import tpu_sc as plsc

NC = 2
NS = 16
NW = NC * NS
SUB = 80


def _node_proj_body(x_ref, w_ref, b_ref, xs1_ref, xd1_ref, xm_ref, hb_ref):
    p = jnp.dot(x_ref[...], w_ref[...], preferred_element_type=jnp.float32)
    xs1_ref[...] = p[:, 0:16]
    xd1_ref[...] = p[:, 16:32]
    xm_ref[0] = p[:, 32:96]
    xm_ref[1] = p[:, 96:160]
    hb_ref[...] = p[:, 160:288] + b_ref[...]


def _node_proj(x, wcat, bself2):
    n, dv = x.shape
    blk = 2000
    return pl.pallas_call(
        _node_proj_body,
        grid=(n // blk,),
        in_specs=[
            pl.BlockSpec((blk, dv), lambda i: (i, 0)),
            pl.BlockSpec(wcat.shape, lambda i: (0, 0)),
            pl.BlockSpec((1, dv), lambda i: (0, 0)),
        ],
        out_specs=[
            pl.BlockSpec((blk, 16), lambda i: (i, 0)),
            pl.BlockSpec((blk, 16), lambda i: (i, 0)),
            pl.BlockSpec((2, blk, 64), lambda i: (0, i, 0)),
            pl.BlockSpec((blk, 128), lambda i: (i, 0)),
        ],
        out_shape=[
            jax.ShapeDtypeStruct((n, 16), jnp.float32),
            jax.ShapeDtypeStruct((n, 16), jnp.float32),
            jax.ShapeDtypeStruct((2, n, 64), jnp.float32),
            jax.ShapeDtypeStruct((n, 128), jnp.float32),
        ],
    )(x, wcat, bself2)


def _sc_gather_make(n, e):
    rows = e // SUB
    jrows = 8
    nblk = rows // jrows
    base_blk = nblk // NW
    extra = nblk - base_blk * NW
    blk_edges = jrows * SUB
    mesh = plsc.VectorSubcoreMesh(core_axis_name="c", subcore_axis_name="s")

    @functools.partial(
        pl.kernel,
        out_type=jax.ShapeDtypeStruct((e, 16), jnp.float32),
        mesh=mesh,
        scratch_types=[
            pltpu.VMEM((jrows, SUB), jnp.int32),
            pltpu.VMEM((jrows, SUB), jnp.int32),
            pltpu.VMEM((blk_edges, 16), jnp.float32),
            pltpu.VMEM((blk_edges, 16), jnp.float32),
            pltpu.SemaphoreType.DMA,
        ],
        compiler_params=pltpu.CompilerParams(use_tc_tiling_on_sc=False),
    )
    def k(xs1_hbm, xd1_hbm, src2_hbm, dst2_hbm, out_hbm,
          idxs_v, idxd_v, g1_v, g2_v, sem):
        wid = lax.axis_index("c") * NS + lax.axis_index("s")
        nb = base_blk + jnp.where(wid < extra, 1, 0)

        def chunk(bi, carry):
            r0 = (bi * NW + wid) * jrows
            pltpu.sync_copy(src2_hbm.at[pl.ds(r0, jrows)], idxs_v)
            pltpu.sync_copy(dst2_hbm.at[pl.ds(r0, jrows)], idxd_v)
            cps = []
            for j in range(jrows):
                cps.append(pltpu.async_copy(
                    xs1_hbm.at[idxs_v.at[j]],
                    g1_v.at[pl.ds(j * SUB, SUB)], sem))
                cps.append(pltpu.async_copy(
                    xd1_hbm.at[idxd_v.at[j]],
                    g2_v.at[pl.ds(j * SUB, SUB)], sem))
            for cp in cps:
                cp.wait()

            def addrow(i, c2):
                g1_v[i, :] = g1_v[i, :] + g2_v[i, :]
                return c2
            lax.fori_loop(0, blk_edges, addrow, 0)
            pltpu.sync_copy(g1_v, out_hbm.at[pl.ds(r0 * SUB, blk_edges)])
            return carry

        lax.fori_loop(0, nb, chunk, 0)

    return k


def _edge_mlp_body(gs_ref, ea_ref, w1_ref, b1_ref, w2_ref, b2_ref,
                   wm_ref, bm_ref, e_ref, t_ref):
    pre = (gs_ref[...]
           + jnp.dot(ea_ref[...], w1_ref[...], preferred_element_type=jnp.float32)
           + b1_ref[...])
    r = jnp.maximum(pre, 0.0)
    e = jnp.dot(r, w2_ref[...], preferred_element_type=jnp.float32) + b2_ref[...]
    e_ref[...] = e
    t = jnp.dot(e, wm_ref[...], preferred_element_type=jnp.float32) + bm_ref[...]
    t_ref[0] = t[:, :64]
    t_ref[1] = t[:, 64:]


def _edge_mlp(gsum, ea, w1, b1, w2, b2, wm, bm):
    e_n, de = ea.shape
    blk = 4000
    return pl.pallas_call(
        _edge_mlp_body,
        grid=(e_n // blk,),
        in_specs=[
            pl.BlockSpec((blk, 16), lambda i: (i, 0)),
            pl.BlockSpec((blk, de), lambda i: (i, 0)),
            pl.BlockSpec(w1.shape, lambda i: (0, 0)),
            pl.BlockSpec((1, 16), lambda i: (0, 0)),
            pl.BlockSpec(w2.shape, lambda i: (0, 0)),
            pl.BlockSpec((1, 16), lambda i: (0, 0)),
            pl.BlockSpec(wm.shape, lambda i: (0, 0)),
            pl.BlockSpec((1, 128), lambda i: (0, 0)),
        ],
        out_specs=[
            pl.BlockSpec((blk, 16), lambda i: (i, 0)),
            pl.BlockSpec((2, blk, 64), lambda i: (0, i, 0)),
        ],
        out_shape=[
            jax.ShapeDtypeStruct((e_n, 16), jnp.float32),
            jax.ShapeDtypeStruct((2, e_n, 64), jnp.float32),
        ],
    )(gsum, ea, w1, b1, w2, b2, wm, bm)


def _sc_msg_make(n_pad, e, emit_m=False):
    rows = e // SUB
    jrows = 8
    nblk = rows // jrows
    base_blk = nblk // NS
    extra = nblk - base_blk * NS
    blk_edges = jrows * SUB
    npc = n_pad // NS
    mesh = plsc.VectorSubcoreMesh(core_axis_name="c", subcore_axis_name="s")

    @functools.partial(
        pl.kernel,
        out_type=([jax.ShapeDtypeStruct((NC, n_pad, 64), jnp.float32),
                   jax.ShapeDtypeStruct((NC, e, 64), jnp.float32)]
                  if emit_m else
                  jax.ShapeDtypeStruct((NC, n_pad, 64), jnp.float32)),
        mesh=mesh,
        scratch_types=[
            pltpu.VMEM((jrows, SUB), jnp.int32),
            pltpu.VMEM((jrows, SUB), jnp.int32),
            pltpu.VMEM((blk_edges, 64), jnp.float32),
            pltpu.VMEM((blk_edges, 64), jnp.float32),
            pltpu.VMEM_SHARED((n_pad, 64), jnp.float32),
            pltpu.SemaphoreType.DMA,
        ],
        compiler_params=pltpu.CompilerParams(use_tc_tiling_on_sc=False),
    )
    def k(xmcat_hbm, t2_hbm, srcb_hbm, dst2_hbm, *rest):
        if emit_m:
            out_hbm, m_hbm = rest[0], rest[1]
            idxs_v, idxd_v, g_v, t_v, agg_sp, sem = rest[2:]
        else:
            out_hbm = rest[0]
            m_hbm = None
            idxs_v, idxd_v, g_v, t_v, agg_sp, sem = rest[1:]
        c = lax.axis_index("c")
        s = lax.axis_index("s")

        def zrow(i, c2):
            for jj in range(4):
                g_v[i, pl.ds(jj * 16, 16)] = jnp.zeros((16,), jnp.float32)
            return c2
        lax.fori_loop(0, blk_edges, zrow, 0)
        for z in range((npc + blk_edges - 1) // blk_edges):
            size = min(blk_edges, npc - z * blk_edges)
            pltpu.sync_copy(
                g_v.at[pl.ds(0, size)],
                agg_sp.at[pl.ds(s * npc + z * blk_edges, size)])
        plsc.subcore_barrier()
        nb = base_blk + jnp.where(s < extra, 1, 0)

        def chunk(bi, carry):
            r0 = (bi * NS + s) * jrows
            pltpu.sync_copy(srcb_hbm.at[pl.ds(c * rows + r0, jrows)], idxs_v)
            pltpu.sync_copy(dst2_hbm.at[pl.ds(r0, jrows)], idxd_v)
            cps = [pltpu.async_copy(xmcat_hbm.at[idxs_v.at[j]],
                                    g_v.at[pl.ds(j * SUB, SUB)], sem)
                   for j in range(jrows)]
            pltpu.sync_copy(t2_hbm.at[c, pl.ds(r0 * SUB, blk_edges)], t_v)
            for cp in cps:
                cp.wait()

            def addrow(i, c2):
                for jj in range(4):
                    sl = pl.ds(jj * 16, 16)
                    g_v[i, sl] = jnp.maximum(g_v[i, sl] + t_v[i, sl], 0.0)
                return c2
            lax.fori_loop(0, blk_edges, addrow, 0)
            if emit_m:
                pltpu.sync_copy(g_v, m_hbm.at[c, pl.ds(r0 * SUB, blk_edges)])
            for j in range(jrows):
                pltpu.sync_copy(g_v.at[pl.ds(j * SUB, SUB)],
                                agg_sp.at[idxd_v.at[j]], add=True)
            return carry

        lax.fori_loop(0, nb, chunk, 0)
        plsc.subcore_barrier()
        pltpu.sync_copy(agg_sp.at[pl.ds(s * npc, npc)],
                        out_hbm.at[c, pl.ds(s * npc, npc)])

    return k


def _finish_body(hb_ref, a0_ref, a1_ref, h_ref):
    agg = jnp.concatenate([a0_ref[...], a1_ref[...]], axis=1)
    h_ref[...] = jnp.maximum(hb_ref[...] + agg, 0.0)


def _finish(hbase, a0, a1):
    n, dv = hbase.shape
    blk = 2000
    spec = pl.BlockSpec((blk, dv), lambda i: (i, 0))
    hspec = pl.BlockSpec((blk, dv // 2), lambda i: (i, 0))
    return pl.pallas_call(
        _finish_body,
        grid=(n // blk,),
        in_specs=[spec, hspec, hspec],
        out_specs=spec,
        out_shape=jax.ShapeDtypeStruct((n, dv), jnp.float32),
    )(hbase, a0, a1)


def kernel(x, edge_index, edge_attr, We1, be1, We2, be2,
           Wmsg, bmsg, Wself, bself):
    n, dv = x.shape
    e = edge_index.shape[1]
    src2 = edge_index[0].reshape(e // SUB, SUB)
    dst2 = edge_index[1].reshape(e // SUB, SUB)

    wcat = jnp.concatenate(
        [We1[:dv], We1[dv:2 * dv], Wmsg[:dv], Wself], axis=1)
    xs1, xd1, xm2, hbase = _node_proj(x, wcat, bself[None, :])
    xmcat = xm2.reshape(2 * n, 64)
    srcb = jnp.concatenate([src2, src2 + n], axis=0)

    gsum = _sc_gather_make(n, e)(xs1, xd1, src2, dst2)
    e_out, t2 = _edge_mlp(gsum, edge_attr, We1[2 * dv:], be1[None, :],
                          We2, be2[None, :], Wmsg[dv:], bmsg[None, :])
    n_pad = ((n + NS * 8 - 1) // (NS * 8)) * NS * 8
    agg2 = _sc_msg_make(n_pad, e)(xmcat, t2, srcb, dst2)
    h = _finish(hbase, agg2[0, :n], agg2[1, :n])
    return (h, e_out)
